# Initial kernel scaffold; baseline (speedup 1.0000x reference)
import jax
import jax.numpy as jnp
from jax.experimental import pallas as pl


def kernel(x, edge_index, batch, A, Wp, bp, W0a, W1a, ba, W0b, W1b, bb, g1, be1, g2, be2, Wl1, bl1, Wl2, bl2, Wl3, bl3, g5, be5, g6, be6):
    raise NotImplementedError("write your pallas kernel here")


# speedup vs baseline: 1.1327x; 1.1327x over previous
import jax
import jax.numpy as jnp
from jax import lax
from jax.experimental import pallas as pl

_N = 10000
_E = 78000
_G = 64
_MB = 1000
_NEG = -3.0e38


def _mm_body(x_ref, w_ref, o_ref):
    o_ref[...] = jnp.dot(x_ref[...], w_ref[...], preferred_element_type=jnp.float32)


def _matmul(x, w):
    m, k = x.shape
    n = w.shape[1]
    return pl.pallas_call(
        _mm_body,
        grid=(m // _MB,),
        in_specs=[pl.BlockSpec((_MB, k), lambda i: (i, 0)),
                  pl.BlockSpec((k, n), lambda i: (0, 0))],
        out_specs=pl.BlockSpec((_MB, n), lambda i: (i, 0)),
        out_shape=jax.ShapeDtypeStruct((m, n), jnp.float32),
    )(x, w)


def _layer2_body(h0_ref, agg_ref, b_ref, g_ref, be_ref, w_ref, o_ref):
    h = h0_ref[...] + agg_ref[...] + b_ref[...]
    h = jnp.maximum(h * g_ref[...] + be_ref[...], 0.0)
    o_ref[...] = jnp.dot(h, w_ref[...], preferred_element_type=jnp.float32)


def _layer2(h0, agg, b, g, be, wcat):
    return pl.pallas_call(
        _layer2_body,
        grid=(_N // _MB,),
        in_specs=[pl.BlockSpec((_MB, 512), lambda i: (i, 0)),
                  pl.BlockSpec((_MB, 512), lambda i: (i, 0)),
                  pl.BlockSpec((1, 512), lambda i: (0, 0)),
                  pl.BlockSpec((1, 512), lambda i: (0, 0)),
                  pl.BlockSpec((1, 512), lambda i: (0, 0)),
                  pl.BlockSpec((512, 1024), lambda i: (0, 0))],
        out_specs=pl.BlockSpec((_MB, 1024), lambda i: (i, 0)),
        out_shape=jax.ShapeDtypeStruct((_N, 1024), jnp.float32),
    )(h0, agg, b, g, be, wcat)


def _pool_body(h0_ref, agg_ref, b_ref, g_ref, be_ref, batch_ref, bcol_ref,
               sum_ref, max_ref, cnt_ref):
    i = pl.program_id(0)

    @pl.when(i == 0)
    def _():
        sum_ref[...] = jnp.zeros_like(sum_ref)
        max_ref[...] = jnp.full_like(max_ref, _NEG)
        cnt_ref[...] = jnp.zeros_like(cnt_ref)

    h = h0_ref[...] + agg_ref[...] + b_ref[...]
    h = jnp.maximum(h * g_ref[...] + be_ref[...], 0.0)
    b = batch_ref[0]
    gids = lax.broadcasted_iota(jnp.int32, (_G, _MB), 0)
    onehot = (gids == b).astype(jnp.float32)
    sum_ref[...] += lax.dot_general(onehot, h, (((1,), (0,)), ((), ())),
                                    preferred_element_type=jnp.float32)
    cnt_ref[...] += jnp.sum(onehot, axis=1, keepdims=True)

    bcol = bcol_ref[...]

    def body(g, _):
        m = bcol == g

        @pl.when(jnp.any(m))
        def _():
            hm = jnp.where(m, h, _NEG)
            mg = jnp.max(hm, axis=0)
            max_ref[pl.ds(g, 1), :] = jnp.maximum(max_ref[pl.ds(g, 1), :],
                                                  mg[None, :])
        return 0

    lax.fori_loop(0, _G, body, 0)


def _pool(h0, agg, b, g, be, batch3, batchcol):
    return pl.pallas_call(
        _pool_body,
        grid=(_N // _MB,),
        in_specs=[pl.BlockSpec((_MB, 512), lambda i: (i, 0)),
                  pl.BlockSpec((_MB, 512), lambda i: (i, 0)),
                  pl.BlockSpec((1, 512), lambda i: (0, 0)),
                  pl.BlockSpec((1, 512), lambda i: (0, 0)),
                  pl.BlockSpec((1, 512), lambda i: (0, 0)),
                  pl.BlockSpec((1, 1, _MB), lambda i: (i, 0, 0)),
                  pl.BlockSpec((_MB, 1), lambda i: (i, 0))],
        out_specs=[pl.BlockSpec((_G, 512), lambda i: (0, 0)),
                   pl.BlockSpec((_G, 512), lambda i: (0, 0)),
                   pl.BlockSpec((_G, 1), lambda i: (0, 0))],
        out_shape=[jax.ShapeDtypeStruct((_G, 512), jnp.float32),
                   jax.ShapeDtypeStruct((_G, 512), jnp.float32),
                   jax.ShapeDtypeStruct((_G, 1), jnp.float32)],
    )(h0, agg, b, g, be, batch3, batchcol)


def _head_body(s1_ref, m1_ref, s2_ref, m2_ref, cnt_ref,
               w1a_ref, w1b_ref, b1_ref, g5_ref, be5_ref,
               w2_ref, b2_ref, g6_ref, be6_ref, w3_ref, b3_ref,
               o1_ref, o2_ref, oo_ref, f1_ref, f2_ref):
    cnt = cnt_ref[...]

    def branch(s_ref, m_ref):
        mean = s_ref[...] / jnp.maximum(cnt, 1.0)
        mx = jnp.where(cnt > 0.0, m_ref[...], 0.0)
        f = jnp.dot(mx, w1a_ref[...], preferred_element_type=jnp.float32)
        f += jnp.dot(mean, w1b_ref[...], preferred_element_type=jnp.float32)
        f = jnp.maximum((f + b1_ref[...]) * g5_ref[...] + be5_ref[...], 0.0)
        f = jnp.dot(f, w2_ref[...], preferred_element_type=jnp.float32)
        f = jnp.maximum((f + b2_ref[...]) * g6_ref[...] + be6_ref[...], 0.0)
        lg = jnp.dot(f, w3_ref[...], preferred_element_type=jnp.float32)
        lg = jnp.maximum(lg + b3_ref[...], 0.0)
        return lg, f

    lg1, f1 = branch(s1_ref, m1_ref)
    lg2, f2 = branch(s2_ref, m2_ref)

    valid = lax.broadcasted_iota(jnp.int32, lg1.shape, 1) < 4

    def smax(lg):
        xm = jnp.where(valid, lg, _NEG)
        r = jnp.max(xm, axis=1, keepdims=True)
        e = jnp.where(valid, jnp.exp(xm - r), 0.0)
        return e / jnp.sum(e, axis=1, keepdims=True)

    o1_ref[...] = smax(lg1)
    o2_ref[...] = smax(lg2)
    oo_ref[...] = smax(0.5 * (lg1 + lg2))
    f1_ref[...] = f1
    f2_ref[...] = f2


def _head(s1, m1, s2, m2, cnt, w1a, w1b, b1, g5, be5, w2, b2, g6, be6, w3, b3):
    c = lambda shape: pl.BlockSpec(shape, lambda: (0,) * len(shape))
    return pl.pallas_call(
        _head_body,
        in_specs=[c((_G, 512)), c((_G, 512)), c((_G, 512)), c((_G, 512)),
                  c((_G, 1)),
                  c((512, 512)), c((512, 512)), c((1, 512)), c((1, 512)),
                  c((1, 512)),
                  c((512, 256)), c((1, 256)), c((1, 256)), c((1, 256)),
                  c((256, 128)), c((1, 128))],
        out_specs=[c((_G, 128)), c((_G, 128)), c((_G, 128)),
                   c((_G, 256)), c((_G, 256))],
        out_shape=[jax.ShapeDtypeStruct((_G, 128), jnp.float32),
                   jax.ShapeDtypeStruct((_G, 128), jnp.float32),
                   jax.ShapeDtypeStruct((_G, 128), jnp.float32),
                   jax.ShapeDtypeStruct((_G, 256), jnp.float32),
                   jax.ShapeDtypeStruct((_G, 256), jnp.float32)],
    )(s1, m1, s2, m2, cnt, w1a, w1b, b1, g5, be5, w2, b2, g6, be6, w3, b3)


def _branch_agg(s, d, table):
    deg = jnp.zeros((_N,), jnp.float32).at[d].add(1.0)
    invs = lax.rsqrt(jnp.maximum(deg, 1.0))
    norm = -(invs[s] * invs[d])
    return jnp.zeros((_N, table.shape[1]), jnp.float32).at[d].add(
        table[s] * norm[:, None])


def kernel(x, edge_index, batch, A, Wp, bp, W0a, W1a, ba, W0b, W1b, bb,
           g1, be1, g2, be2, Wl1, bl1, Wl2, bl2, Wl3, bl3, g5, be5, g6, be6):
    f32 = jnp.float32
    r2 = lambda v: v.reshape(1, -1)

    wcat = jnp.concatenate([W0a, W1a, Wp[:1025], Wp[1025:]], axis=1)
    wcat = jnp.pad(wcat, ((0, 7), (0, 1152 - 1026)))
    xp = jnp.pad(x, ((0, 0), (0, 7)))
    big = _matmul(xp, wcat)
    h0 = big[:, :512]
    xw1 = big[:, 512:1024]
    p = big[:, 1024]
    q = big[:, 1025]

    src, dst = edge_index[0], edge_index[1]
    z = p[src] + q[dst]
    order = jnp.argsort(z)
    half = _E // 2
    e2 = order[:half]
    e1 = order[half:]

    batch3 = batch.astype(jnp.int32).reshape(_N // _MB, 1, _MB)
    batchcol = batch.astype(jnp.int32).reshape(_N, 1)
    w2cat = jnp.concatenate([W0b, W1b], axis=1)
    w3p = jnp.pad(Wl3, ((0, 0), (0, 124)))
    b3p = jnp.pad(bl3, (0, 124)).reshape(1, 128)

    def branch_pool(e):
        s, d = src[e], dst[e]
        agg1 = _branch_agg(s, d, xw1)
        l2 = _layer2(h0, agg1, r2(ba), r2(g1), r2(be1), w2cat)
        h0b, hw1b = l2[:, :512], l2[:, 512:]
        agg2 = _branch_agg(s, d, hw1b)
        return _pool(h0b, agg2, r2(bb), r2(g2), r2(be2), batch3, batchcol)

    s1, m1, cnt = branch_pool(e1)
    s2, m2, _ = branch_pool(e2)

    o1, o2, oo, f1, f2 = _head(
        s1, m1, s2, m2, cnt,
        Wl1[:512], Wl1[512:], r2(bl1), r2(g5), r2(be5),
        Wl2, r2(bl2), r2(g6), r2(be6), w3p, b3p)

    return (o1[:, :4], o2[:, :4], oo[:, :4],
            jnp.concatenate([f1, f2], axis=1))


# speedup vs baseline: 1.3932x; 1.2301x over previous
import functools

import jax
import jax.numpy as jnp
from jax import lax
from jax.experimental import pallas as pl
from jax.experimental.pallas import tpu as pltpu
from jax.experimental.pallas import tpu_sc as plsc

_N = 10000
_E = 78000
_G = 64
_MB = 1000
_NEG = -3.0e38

_NP = 10240
_EP = 40960
_CH = 128
_RPT = _NP // 16


def _mm_body(x_ref, w_ref, o_ref):
    o_ref[...] = jnp.dot(x_ref[...], w_ref[...], preferred_element_type=jnp.float32)


def _matmul(x, w):
    m, k = x.shape
    n = w.shape[1]
    return pl.pallas_call(
        _mm_body,
        grid=(m // _MB,),
        in_specs=[pl.BlockSpec((_MB, k), lambda i: (i, 0)),
                  pl.BlockSpec((k, n), lambda i: (0, 0))],
        out_specs=pl.BlockSpec((_MB, n), lambda i: (i, 0)),
        out_shape=jax.ShapeDtypeStruct((m, n), jnp.float32),
    )(x, w)


def _layer2_body(h0_ref, agg_ref, b_ref, g_ref, be_ref, w_ref, o_ref):
    h = h0_ref[...] + agg_ref[...] + b_ref[...]
    h = jnp.maximum(h * g_ref[...] + be_ref[...], 0.0)
    o_ref[...] = jnp.dot(h, w_ref[...], preferred_element_type=jnp.float32)


def _layer2(h0, agg, b, g, be, wcat):
    return pl.pallas_call(
        _layer2_body,
        grid=(_N // _MB,),
        in_specs=[pl.BlockSpec((_MB, 512), lambda i: (i, 0)),
                  pl.BlockSpec((_MB, 512), lambda i: (i, 0)),
                  pl.BlockSpec((1, 512), lambda i: (0, 0)),
                  pl.BlockSpec((1, 512), lambda i: (0, 0)),
                  pl.BlockSpec((1, 512), lambda i: (0, 0)),
                  pl.BlockSpec((512, 1024), lambda i: (0, 0))],
        out_specs=pl.BlockSpec((_MB, 1024), lambda i: (i, 0)),
        out_shape=jax.ShapeDtypeStruct((_N, 1024), jnp.float32),
    )(h0, agg, b, g, be, wcat)


def _pool_body(h0_ref, agg_ref, b_ref, g_ref, be_ref, batch_ref, bcol_ref,
               sum_ref, max_ref, cnt_ref):
    i = pl.program_id(0)

    @pl.when(i == 0)
    def _():
        sum_ref[...] = jnp.zeros_like(sum_ref)
        max_ref[...] = jnp.full_like(max_ref, _NEG)
        cnt_ref[...] = jnp.zeros_like(cnt_ref)

    h = h0_ref[...] + agg_ref[...] + b_ref[...]
    h = jnp.maximum(h * g_ref[...] + be_ref[...], 0.0)
    b = batch_ref[0]
    gids = lax.broadcasted_iota(jnp.int32, (_G, _MB), 0)
    onehot = (gids == b).astype(jnp.float32)
    sum_ref[...] += lax.dot_general(onehot, h, (((1,), (0,)), ((), ())),
                                    preferred_element_type=jnp.float32)
    cnt_ref[...] += jnp.sum(onehot, axis=1, keepdims=True)

    bcol = bcol_ref[...]

    def body(g, _):
        m = bcol == g

        @pl.when(jnp.any(m))
        def _():
            hm = jnp.where(m, h, _NEG)
            mg = jnp.max(hm, axis=0)
            max_ref[pl.ds(g, 1), :] = jnp.maximum(max_ref[pl.ds(g, 1), :],
                                                  mg[None, :])
        return 0

    lax.fori_loop(0, _G, body, 0)


def _pool(h0, agg, b, g, be, batch3, batchcol):
    return pl.pallas_call(
        _pool_body,
        grid=(_N // _MB,),
        in_specs=[pl.BlockSpec((_MB, 512), lambda i: (i, 0)),
                  pl.BlockSpec((_MB, 512), lambda i: (i, 0)),
                  pl.BlockSpec((1, 512), lambda i: (0, 0)),
                  pl.BlockSpec((1, 512), lambda i: (0, 0)),
                  pl.BlockSpec((1, 512), lambda i: (0, 0)),
                  pl.BlockSpec((1, 1, _MB), lambda i: (i, 0, 0)),
                  pl.BlockSpec((_MB, 1), lambda i: (i, 0))],
        out_specs=[pl.BlockSpec((_G, 512), lambda i: (0, 0)),
                   pl.BlockSpec((_G, 512), lambda i: (0, 0)),
                   pl.BlockSpec((_G, 1), lambda i: (0, 0))],
        out_shape=[jax.ShapeDtypeStruct((_G, 512), jnp.float32),
                   jax.ShapeDtypeStruct((_G, 512), jnp.float32),
                   jax.ShapeDtypeStruct((_G, 1), jnp.float32)],
    )(h0, agg, b, g, be, batch3, batchcol)


def _head_body(s1_ref, m1_ref, s2_ref, m2_ref, cnt_ref,
               w1a_ref, w1b_ref, b1_ref, g5_ref, be5_ref,
               w2_ref, b2_ref, g6_ref, be6_ref, w3_ref, b3_ref,
               o1_ref, o2_ref, oo_ref, f1_ref, f2_ref):
    cnt = cnt_ref[...]

    def branch(s_ref, m_ref):
        mean = s_ref[...] / jnp.maximum(cnt, 1.0)
        mx = jnp.where(cnt > 0.0, m_ref[...], 0.0)
        f = jnp.dot(mx, w1a_ref[...], preferred_element_type=jnp.float32)
        f += jnp.dot(mean, w1b_ref[...], preferred_element_type=jnp.float32)
        f = jnp.maximum((f + b1_ref[...]) * g5_ref[...] + be5_ref[...], 0.0)
        f = jnp.dot(f, w2_ref[...], preferred_element_type=jnp.float32)
        f = jnp.maximum((f + b2_ref[...]) * g6_ref[...] + be6_ref[...], 0.0)
        lg = jnp.dot(f, w3_ref[...], preferred_element_type=jnp.float32)
        lg = jnp.maximum(lg + b3_ref[...], 0.0)
        return lg, f

    lg1, f1 = branch(s1_ref, m1_ref)
    lg2, f2 = branch(s2_ref, m2_ref)

    valid = lax.broadcasted_iota(jnp.int32, lg1.shape, 1) < 4

    def smax(lg):
        xm = jnp.where(valid, lg, _NEG)
        r = jnp.max(xm, axis=1, keepdims=True)
        e = jnp.where(valid, jnp.exp(xm - r), 0.0)
        return e / jnp.sum(e, axis=1, keepdims=True)

    o1_ref[...] = smax(lg1)
    o2_ref[...] = smax(lg2)
    oo_ref[...] = smax(0.5 * (lg1 + lg2))
    f1_ref[...] = f1
    f2_ref[...] = f2


def _head(s1, m1, s2, m2, cnt, w1a, w1b, b1, g5, be5, w2, b2, g6, be6, w3, b3):
    c = lambda shape: pl.BlockSpec(shape, lambda: (0,) * len(shape))
    return pl.pallas_call(
        _head_body,
        in_specs=[c((_G, 512)), c((_G, 512)), c((_G, 512)), c((_G, 512)),
                  c((_G, 1)),
                  c((512, 512)), c((512, 512)), c((1, 512)), c((1, 512)),
                  c((1, 512)),
                  c((512, 256)), c((1, 256)), c((1, 256)), c((1, 256)),
                  c((256, 128)), c((1, 128))],
        out_specs=[c((_G, 128)), c((_G, 128)), c((_G, 128)),
                   c((_G, 256)), c((_G, 256))],
        out_shape=[jax.ShapeDtypeStruct((_G, 128), jnp.float32),
                   jax.ShapeDtypeStruct((_G, 128), jnp.float32),
                   jax.ShapeDtypeStruct((_G, 128), jnp.float32),
                   jax.ShapeDtypeStruct((_G, 256), jnp.float32),
                   jax.ShapeDtypeStruct((_G, 256), jnp.float32)],
    )(s1, m1, s2, m2, cnt, w1a, w1b, b1, g5, be5, w2, b2, g6, be6, w3, b3)


def _agg_sc_call(t0, t1, t2, t3, srcp, dstp, nrmp, zblk):
    mesh = plsc.VectorSubcoreMesh(core_axis_name="c", subcore_axis_name="s")

    @functools.partial(
        pl.kernel, mesh=mesh,
        out_type=jax.ShapeDtypeStruct((_NP, 512), jnp.float32),
        scratch_types=[
            pltpu.VMEM((_CH,), jnp.int32),
            pltpu.VMEM((_CH,), jnp.int32),
            pltpu.VMEM((_CH, 16), jnp.float32),
            pltpu.VMEM((_CH, 128), jnp.float32),
            pltpu.VMEM_SHARED((_NP, 128), jnp.float32),
            pltpu.SemaphoreType.DMA,
        ])
    def k(t0h, t1h, t2h, t3h, sh, dh, nh, zh, outh, sv, dv, nv, rows, acc, sem):
        c = lax.axis_index("c")
        s = lax.axis_index("s")
        tables = (t0h, t1h, t2h, t3h)

        def run_chunk(cs, th):
            pltpu.sync_copy(zh, acc.at[pl.ds(s * _RPT, _RPT)])
            plsc.subcore_barrier()

            def chunk_body(i, carry):
                base = s * (_EP // 16) + i * _CH
                pltpu.sync_copy(sh.at[pl.ds(base, _CH)], sv)
                pltpu.sync_copy(dh.at[pl.ds(base, _CH)], dv)
                pltpu.sync_copy(nh.at[pl.ds(base, _CH)], nv)
                pltpu.async_copy(th.at[sv], rows, sem).wait()

                def row_body(r, c2):
                    nsp = nv[r, :]
                    for j in range(8):
                        rows[r, pl.ds(j * 16, 16)] = \
                            rows[r, pl.ds(j * 16, 16)] * nsp
                    return c2

                lax.fori_loop(0, _CH, row_body, 0)
                pltpu.sync_copy(rows, acc.at[dv], add=True)
                return carry

            lax.fori_loop(0, _EP // 16 // _CH, chunk_body, 0)
            plsc.subcore_barrier()
            r0 = s * _RPT
            pltpu.sync_copy(acc.at[pl.ds(r0, _RPT)],
                            outh.at[pl.ds(r0, _RPT), pl.ds(cs * 128, 128)])

        for cv in (0, 1):
            @pl.when(c == cv)
            def _(cv=cv):
                run_chunk(cv * 2, tables[cv * 2])
                run_chunk(cv * 2 + 1, tables[cv * 2 + 1])

    return k(t0, t1, t2, t3, srcp, dstp, nrmp, zblk)


def _branch_agg(s, d, table):
    deg = jnp.zeros((_N,), jnp.float32).at[d].add(1.0)
    invs = lax.rsqrt(jnp.maximum(deg, 1.0))
    nrm = -(invs[s] * invs[d])
    srcp = jnp.pad(s.astype(jnp.int32), (0, _EP - s.shape[0]))
    dstp = jnp.pad(d.astype(jnp.int32), (0, _EP - d.shape[0]),
                   constant_values=_NP - 1)
    nrmp = jnp.broadcast_to(
        jnp.pad(nrm, (0, _EP - nrm.shape[0]))[:, None], (_EP, 16))
    zblk = jnp.zeros((_RPT, 128), jnp.float32)
    t = [table[:, i * 128:(i + 1) * 128] for i in range(4)]
    out = _agg_sc_call(t[0], t[1], t[2], t[3], srcp, dstp, nrmp, zblk)
    return out[:_N]


def kernel(x, edge_index, batch, A, Wp, bp, W0a, W1a, ba, W0b, W1b, bb,
           g1, be1, g2, be2, Wl1, bl1, Wl2, bl2, Wl3, bl3, g5, be5, g6, be6):
    f32 = jnp.float32
    r2 = lambda v: v.reshape(1, -1)

    wcat = jnp.concatenate([W0a, W1a, Wp[:1025], Wp[1025:]], axis=1)
    wcat = jnp.pad(wcat, ((0, 7), (0, 1152 - 1026)))
    xp = jnp.pad(x, ((0, 0), (0, 7)))
    big = _matmul(xp, wcat)
    h0 = big[:, :512]
    xw1 = big[:, 512:1024]
    p = big[:, 1024]
    q = big[:, 1025]

    src, dst = edge_index[0], edge_index[1]
    z = p[src] + q[dst]
    order = jnp.argsort(z)
    half = _E // 2
    e2 = order[:half]
    e1 = order[half:]

    batch3 = batch.astype(jnp.int32).reshape(_N // _MB, 1, _MB)
    batchcol = batch.astype(jnp.int32).reshape(_N, 1)
    w2cat = jnp.concatenate([W0b, W1b], axis=1)
    w3p = jnp.pad(Wl3, ((0, 0), (0, 124)))
    b3p = jnp.pad(bl3, (0, 124)).reshape(1, 128)

    def branch_pool(e):
        s, d = src[e], dst[e]
        agg1 = _branch_agg(s, d, xw1)
        l2 = _layer2(h0, agg1, r2(ba), r2(g1), r2(be1), w2cat)
        h0b, hw1b = l2[:, :512], l2[:, 512:]
        agg2 = _branch_agg(s, d, hw1b)
        return _pool(h0b, agg2, r2(bb), r2(g2), r2(be2), batch3, batchcol)

    s1, m1, cnt = branch_pool(e1)
    s2, m2, _ = branch_pool(e2)

    o1, o2, oo, f1, f2 = _head(
        s1, m1, s2, m2, cnt,
        Wl1[:512], Wl1[512:], r2(bl1), r2(g5), r2(be5),
        Wl2, r2(bl2), r2(g6), r2(be6), w3p, b3p)

    return (o1[:, :4], o2[:, :4], oo[:, :4],
            jnp.concatenate([f1, f2], axis=1))


# speedup vs baseline: 1.4754x; 1.0590x over previous
import functools

import jax
import jax.numpy as jnp
from jax import lax
from jax.experimental import pallas as pl
from jax.experimental.pallas import tpu as pltpu
from jax.experimental.pallas import tpu_sc as plsc

_N = 10000
_E = 78000
_G = 64
_MB = 1000
_NEG = -3.0e38

_NP = 10240
_EP = 40960
_CH = 64
_RPT = _NP // 16


def _mm_body(x_ref, w_ref, o_ref):
    o_ref[...] = jnp.dot(x_ref[...], w_ref[...], preferred_element_type=jnp.float32)


def _matmul(x, w):
    m, k = x.shape
    n = w.shape[1]
    return pl.pallas_call(
        _mm_body,
        grid=(m // _MB,),
        in_specs=[pl.BlockSpec((_MB, k), lambda i: (i, 0)),
                  pl.BlockSpec((k, n), lambda i: (0, 0))],
        out_specs=pl.BlockSpec((_MB, n), lambda i: (i, 0)),
        out_shape=jax.ShapeDtypeStruct((m, n), jnp.float32),
    )(x, w)


def _layer2_body(h0_ref, agg_ref, b_ref, g_ref, be_ref, w_ref, o_ref):
    h = h0_ref[...] + agg_ref[...] + b_ref[...]
    h = jnp.maximum(h * g_ref[...] + be_ref[...], 0.0)
    o_ref[...] = jnp.dot(h, w_ref[...], preferred_element_type=jnp.float32)


def _layer2(h0, agg, b, g, be, wcat):
    return pl.pallas_call(
        _layer2_body,
        grid=(_N // _MB,),
        in_specs=[pl.BlockSpec((_MB, 512), lambda i: (i, 0)),
                  pl.BlockSpec((_MB, 512), lambda i: (i, 0)),
                  pl.BlockSpec((1, 512), lambda i: (0, 0)),
                  pl.BlockSpec((1, 512), lambda i: (0, 0)),
                  pl.BlockSpec((1, 512), lambda i: (0, 0)),
                  pl.BlockSpec((512, 1024), lambda i: (0, 0))],
        out_specs=pl.BlockSpec((_MB, 1024), lambda i: (i, 0)),
        out_shape=jax.ShapeDtypeStruct((_N, 1024), jnp.float32),
    )(h0, agg, b, g, be, wcat)


def _pool_body(h0_ref, agg_ref, b_ref, g_ref, be_ref, batch_ref, bcol_ref,
               sum_ref, max_ref, cnt_ref):
    i = pl.program_id(0)

    @pl.when(i == 0)
    def _():
        sum_ref[...] = jnp.zeros_like(sum_ref)
        max_ref[...] = jnp.full_like(max_ref, _NEG)
        cnt_ref[...] = jnp.zeros_like(cnt_ref)

    h = h0_ref[...] + agg_ref[...] + b_ref[...]
    h = jnp.maximum(h * g_ref[...] + be_ref[...], 0.0)
    b = batch_ref[0]
    gids = lax.broadcasted_iota(jnp.int32, (_G, _MB), 0)
    onehot = (gids == b).astype(jnp.float32)
    sum_ref[...] += lax.dot_general(onehot, h, (((1,), (0,)), ((), ())),
                                    preferred_element_type=jnp.float32)
    cnt_ref[...] += jnp.sum(onehot, axis=1, keepdims=True)

    bcol = bcol_ref[...]

    def body(g, _):
        m = bcol == g

        @pl.when(jnp.any(m))
        def _():
            hm = jnp.where(m, h, _NEG)
            mg = jnp.max(hm, axis=0)
            max_ref[pl.ds(g, 1), :] = jnp.maximum(max_ref[pl.ds(g, 1), :],
                                                  mg[None, :])
        return 0

    lax.fori_loop(0, _G, body, 0)


def _pool(h0, agg, b, g, be, batch3, batchcol):
    return pl.pallas_call(
        _pool_body,
        grid=(_N // _MB,),
        in_specs=[pl.BlockSpec((_MB, 512), lambda i: (i, 0)),
                  pl.BlockSpec((_MB, 512), lambda i: (i, 0)),
                  pl.BlockSpec((1, 512), lambda i: (0, 0)),
                  pl.BlockSpec((1, 512), lambda i: (0, 0)),
                  pl.BlockSpec((1, 512), lambda i: (0, 0)),
                  pl.BlockSpec((1, 1, _MB), lambda i: (i, 0, 0)),
                  pl.BlockSpec((_MB, 1), lambda i: (i, 0))],
        out_specs=[pl.BlockSpec((_G, 512), lambda i: (0, 0)),
                   pl.BlockSpec((_G, 512), lambda i: (0, 0)),
                   pl.BlockSpec((_G, 1), lambda i: (0, 0))],
        out_shape=[jax.ShapeDtypeStruct((_G, 512), jnp.float32),
                   jax.ShapeDtypeStruct((_G, 512), jnp.float32),
                   jax.ShapeDtypeStruct((_G, 1), jnp.float32)],
    )(h0, agg, b, g, be, batch3, batchcol)


def _head_body(s1_ref, m1_ref, s2_ref, m2_ref, cnt_ref,
               w1a_ref, w1b_ref, b1_ref, g5_ref, be5_ref,
               w2_ref, b2_ref, g6_ref, be6_ref, w3_ref, b3_ref,
               o1_ref, o2_ref, oo_ref, f1_ref, f2_ref):
    cnt = cnt_ref[...]

    def branch(s_ref, m_ref):
        mean = s_ref[...] / jnp.maximum(cnt, 1.0)
        mx = jnp.where(cnt > 0.0, m_ref[...], 0.0)
        f = jnp.dot(mx, w1a_ref[...], preferred_element_type=jnp.float32)
        f += jnp.dot(mean, w1b_ref[...], preferred_element_type=jnp.float32)
        f = jnp.maximum((f + b1_ref[...]) * g5_ref[...] + be5_ref[...], 0.0)
        f = jnp.dot(f, w2_ref[...], preferred_element_type=jnp.float32)
        f = jnp.maximum((f + b2_ref[...]) * g6_ref[...] + be6_ref[...], 0.0)
        lg = jnp.dot(f, w3_ref[...], preferred_element_type=jnp.float32)
        lg = jnp.maximum(lg + b3_ref[...], 0.0)
        return lg, f

    lg1, f1 = branch(s1_ref, m1_ref)
    lg2, f2 = branch(s2_ref, m2_ref)

    valid = lax.broadcasted_iota(jnp.int32, lg1.shape, 1) < 4

    def smax(lg):
        xm = jnp.where(valid, lg, _NEG)
        r = jnp.max(xm, axis=1, keepdims=True)
        e = jnp.where(valid, jnp.exp(xm - r), 0.0)
        return e / jnp.sum(e, axis=1, keepdims=True)

    o1_ref[...] = smax(lg1)
    o2_ref[...] = smax(lg2)
    oo_ref[...] = smax(0.5 * (lg1 + lg2))
    f1_ref[...] = f1
    f2_ref[...] = f2


def _head(s1, m1, s2, m2, cnt, w1a, w1b, b1, g5, be5, w2, b2, g6, be6, w3, b3):
    c = lambda shape: pl.BlockSpec(shape, lambda: (0,) * len(shape))
    return pl.pallas_call(
        _head_body,
        in_specs=[c((_G, 512)), c((_G, 512)), c((_G, 512)), c((_G, 512)),
                  c((_G, 1)),
                  c((512, 512)), c((512, 512)), c((1, 512)), c((1, 512)),
                  c((1, 512)),
                  c((512, 256)), c((1, 256)), c((1, 256)), c((1, 256)),
                  c((256, 128)), c((1, 128))],
        out_specs=[c((_G, 128)), c((_G, 128)), c((_G, 128)),
                   c((_G, 256)), c((_G, 256))],
        out_shape=[jax.ShapeDtypeStruct((_G, 128), jnp.float32),
                   jax.ShapeDtypeStruct((_G, 128), jnp.float32),
                   jax.ShapeDtypeStruct((_G, 128), jnp.float32),
                   jax.ShapeDtypeStruct((_G, 256), jnp.float32),
                   jax.ShapeDtypeStruct((_G, 256), jnp.float32)],
    )(s1, m1, s2, m2, cnt, w1a, w1b, b1, g5, be5, w2, b2, g6, be6, w3, b3)


def _agg_sc_call(t0, t1, t2, t3, srcp, dstp, nrmp, zblk):
    mesh = plsc.VectorSubcoreMesh(core_axis_name="c", subcore_axis_name="s")

    ept = _EP // 16
    nch = ept // _CH

    @functools.partial(
        pl.kernel, mesh=mesh,
        out_type=jax.ShapeDtypeStruct((_NP, 512), jnp.float32),
        scratch_types=[
            pltpu.VMEM((ept,), jnp.int32),
            pltpu.VMEM((nch, _CH), jnp.int32),
            pltpu.VMEM((_CH, 16), jnp.float32),
            pltpu.VMEM((_CH, 16), jnp.float32),
            pltpu.VMEM((_CH, 128), jnp.float32),
            pltpu.VMEM((_CH, 128), jnp.float32),
            pltpu.VMEM_SHARED((_NP, 128), jnp.float32),
            pltpu.SemaphoreType.DMA,
            pltpu.SemaphoreType.DMA,
        ])
    def k(t0h, t1h, t2h, t3h, sh, dh, nh, zh, outh,
          sidx, didx, nv0, nv1, rows0, rows1, acc, sem0, sem1):
        c = lax.axis_index("c")
        s = lax.axis_index("s")
        tables = (t0h, t1h, t2h, t3h)
        bufs = (rows0, rows1)
        nbufs = (nv0, nv1)
        sems = (sem0, sem1)

        pltpu.sync_copy(sh.at[pl.ds(s * ept, ept)], sidx)
        pltpu.sync_copy(dh.at[s], didx)

        def gather(th, i):
            h1 = pltpu.async_copy(
                th.at[sidx.at[pl.ds(i * _CH, _CH)]], bufs[i % 2], sems[i % 2])
            h2 = pltpu.async_copy(
                nh.at[pl.ds(s * ept + i * _CH, _CH)], nbufs[i % 2],
                sems[i % 2])
            return h1, h2

        def run_pass(cs, th):
            pltpu.sync_copy(zh, acc.at[pl.ds(s * _RPT, _RPT)])
            plsc.subcore_barrier()
            cps = [None, None]
            cps[0] = gather(th, 0)
            for i in range(nch):
                b = bufs[i % 2]
                nb = nbufs[i % 2]
                cps[i % 2][0].wait()
                cps[i % 2][1].wait()
                if i + 1 < nch:
                    cps[(i + 1) % 2] = gather(th, i + 1)

                def row_body(r, c2, b=b, nb=nb):
                    nsp = nb[r, :]
                    for j in range(8):
                        b[r, pl.ds(j * 16, 16)] = b[r, pl.ds(j * 16, 16)] * nsp
                    return c2

                lax.fori_loop(0, _CH, row_body, 0)
                pltpu.sync_copy(b, acc.at[didx.at[i]], add=True)
            plsc.subcore_barrier()
            r0 = s * _RPT
            pltpu.sync_copy(acc.at[pl.ds(r0, _RPT)],
                            outh.at[pl.ds(r0, _RPT), pl.ds(cs * 128, 128)])

        for cv in (0, 1):
            @pl.when(c == cv)
            def _(cv=cv):
                run_pass(cv * 2, tables[cv * 2])
                run_pass(cv * 2 + 1, tables[cv * 2 + 1])

    return k(t0, t1, t2, t3, srcp, dstp, nrmp, zblk)


def _branch_agg(s, d, table):
    deg = jnp.zeros((_N,), jnp.float32).at[d].add(1.0)
    invs = lax.rsqrt(jnp.maximum(deg, 1.0))
    nrm = -(invs[s] * invs[d])
    srcp = jnp.pad(s.astype(jnp.int32), (0, _EP - s.shape[0]))
    dstp = jnp.pad(d.astype(jnp.int32), (0, _EP - d.shape[0]),
                   constant_values=_NP - 1).reshape(16, _EP // _CH // 16, _CH)
    nrmp = jnp.broadcast_to(
        jnp.pad(nrm, (0, _EP - nrm.shape[0]))[:, None], (_EP, 16))
    zblk = jnp.zeros((_RPT, 128), jnp.float32)
    t = [table[:, i * 128:(i + 1) * 128] for i in range(4)]
    out = _agg_sc_call(t[0], t[1], t[2], t[3], srcp, dstp, nrmp, zblk)
    return out[:_N]


def kernel(x, edge_index, batch, A, Wp, bp, W0a, W1a, ba, W0b, W1b, bb,
           g1, be1, g2, be2, Wl1, bl1, Wl2, bl2, Wl3, bl3, g5, be5, g6, be6):
    f32 = jnp.float32
    r2 = lambda v: v.reshape(1, -1)

    wcat = jnp.concatenate([W0a, W1a, Wp[:1025], Wp[1025:]], axis=1)
    wcat = jnp.pad(wcat, ((0, 7), (0, 1152 - 1026)))
    xp = jnp.pad(x, ((0, 0), (0, 7)))
    big = _matmul(xp, wcat)
    h0 = big[:, :512]
    xw1 = big[:, 512:1024]
    p = big[:, 1024]
    q = big[:, 1025]

    src, dst = edge_index[0], edge_index[1]
    z = p[src] + q[dst]
    order = jnp.argsort(z)
    half = _E // 2
    e2 = order[:half]
    e1 = order[half:]

    batch3 = batch.astype(jnp.int32).reshape(_N // _MB, 1, _MB)
    batchcol = batch.astype(jnp.int32).reshape(_N, 1)
    w2cat = jnp.concatenate([W0b, W1b], axis=1)
    w3p = jnp.pad(Wl3, ((0, 0), (0, 124)))
    b3p = jnp.pad(bl3, (0, 124)).reshape(1, 128)

    def branch_pool(e):
        s, d = src[e], dst[e]
        agg1 = _branch_agg(s, d, xw1)
        l2 = _layer2(h0, agg1, r2(ba), r2(g1), r2(be1), w2cat)
        h0b, hw1b = l2[:, :512], l2[:, 512:]
        agg2 = _branch_agg(s, d, hw1b)
        return _pool(h0b, agg2, r2(bb), r2(g2), r2(be2), batch3, batchcol)

    s1, m1, cnt = branch_pool(e1)
    s2, m2, _ = branch_pool(e2)

    o1, o2, oo, f1, f2 = _head(
        s1, m1, s2, m2, cnt,
        Wl1[:512], Wl1[512:], r2(bl1), r2(g5), r2(be5),
        Wl2, r2(bl2), r2(g6), r2(be6), w3p, b3p)

    return (o1[:, :4], o2[:, :4], oo[:, :4],
            jnp.concatenate([f1, f2], axis=1))
